# Initial kernel scaffold; baseline (speedup 1.0000x reference)
import jax
import jax.numpy as jnp
from jax.experimental import pallas as pl


def kernel(x, edge_curvature, W_g1, b_g1, W_h1, b_h1, W_g2, b_g2, W_h2, b_h2, edge_index):
    raise NotImplementedError("write your pallas kernel here")


# speedup vs baseline: 5.7559x; 5.7559x over previous
import jax
import jax.numpy as jnp
from jax import lax
from jax.experimental import pallas as pl
from jax.experimental.pallas import tpu as pltpu
from jax.experimental.pallas import tpu_sc as plsc

N = 10000
E = 320000
D = 128
H = 64
NP = 10240
NC = 2
NS = 16
NW = NC * NS
L = 16
C = 128
NCHUNK = E // C
RC = C // L
STRIPE = NP // NS
WSTRIPE = NP // NW


def _sigm(v):
    return 1.0 / (1.0 + jnp.exp(v * (-0.2)))


def _deg_body(dst_hbm, curv_hbm, zn_hbm,
              degg_hbm, degh_hbm,
              degg, degh,
              gbuf, hbuf, curvv, dstv, wbuf, wbuf2):
    cidx = lax.axis_index("c")
    sidx = lax.axis_index("s")
    w = sidx * NC + cidx
    base = sidx * STRIPE

    pltpu.sync_copy(zn_hbm, degg.at[pl.ds(base, STRIPE)])
    pltpu.sync_copy(zn_hbm, degh.at[pl.ds(base, STRIPE)])
    plsc.subcore_barrier()

    n_a = jnp.where(sidx < NCHUNK % NS, NCHUNK // NS + 1, NCHUNK // NS)

    def a_body(i, _):
        k = sidx + i * NS
        pltpu.sync_copy(dst_hbm.at[k], dstv)
        pltpu.sync_copy(curv_hbm.at[k], curvv)
        for v in range(RC):
            sl = pl.ds(v * L, L)
            g = _sigm(curvv[sl])
            gbuf[sl] = g
            hbuf[sl] = 1.0 - g
        pltpu.sync_copy(gbuf, degg.at[dstv], add=True)
        pltpu.sync_copy(hbuf, degh.at[dstv], add=True)
        return _

    lax.fori_loop(0, n_a, a_body, None)
    plsc.subcore_barrier()

    wbase = w * WSTRIPE
    pltpu.sync_copy(degg.at[pl.ds(wbase, WSTRIPE)], wbuf)
    pltpu.sync_copy(wbuf, degg_hbm.at[pl.ds(wbase, WSTRIPE)])
    pltpu.sync_copy(degh.at[pl.ds(wbase, WSTRIPE)], wbuf2)
    pltpu.sync_copy(wbuf2, degh_hbm.at[pl.ds(wbase, WSTRIPE)])


def _deg(dst, curv, zn):
    kern = pl.kernel(
        _deg_body,
        out_type=[
            jax.ShapeDtypeStruct((NP,), jnp.float32),
            jax.ShapeDtypeStruct((NP,), jnp.float32),
        ],
        mesh=plsc.VectorSubcoreMesh(core_axis_name="c", subcore_axis_name="s",
                                    num_cores=NC),
        scratch_types=[
            pltpu.VMEM_SHARED((NP,), jnp.float32),
            pltpu.VMEM_SHARED((NP,), jnp.float32),
            pltpu.VMEM((C,), jnp.float32),
            pltpu.VMEM((C,), jnp.float32),
            pltpu.VMEM((C,), jnp.float32),
            pltpu.VMEM((C,), jnp.int32),
            pltpu.VMEM((WSTRIPE,), jnp.float32),
            pltpu.VMEM((WSTRIPE,), jnp.float32),
        ],
    )
    return kern(dst, curv, zn)


def _dinv(degg):
    def body(d_ref, o_ref):
        o_ref[...] = lax.rsqrt(d_ref[...] + 1.0)

    return pl.pallas_call(
        body,
        out_shape=jax.ShapeDtypeStruct((NP // D, D), jnp.float32),
    )(degg.reshape(NP // D, D))


def _norm_body(src_hbm, dst_hbm, curv_hbm, dinv_hbm,
               norm_hbm, ghet_hbm,
               dinv_sp,
               srcv, dstv, curvv, normv, ghetv, dsb, ddb):
    cidx = lax.axis_index("c")
    sidx = lax.axis_index("s")
    w = sidx * NC + cidx
    base = sidx * STRIPE

    pltpu.sync_copy(dinv_hbm.at[pl.ds(base, STRIPE)],
                    dinv_sp.at[pl.ds(base, STRIPE)])
    plsc.subcore_barrier()
    n_c = jnp.where(w < NCHUNK % NW, NCHUNK // NW + 1, NCHUNK // NW)

    def c_body(i, _):
        k = w + i * NW
        pltpu.sync_copy(src_hbm.at[k], srcv)
        pltpu.sync_copy(dst_hbm.at[k], dstv)
        pltpu.sync_copy(curv_hbm.at[k], curvv)
        pltpu.sync_copy(dinv_sp.at[srcv], dsb)
        pltpu.sync_copy(dinv_sp.at[dstv], ddb)
        for v in range(RC):
            sl = pl.ds(v * L, L)
            g = _sigm(curvv[sl])
            normv[sl] = dsb[sl] * g * ddb[sl]
            ghetv[sl] = 1.0 - g
        pltpu.sync_copy(normv, norm_hbm.at[k])
        pltpu.sync_copy(ghetv, ghet_hbm.at[k])
        return _

    lax.fori_loop(0, n_c, c_body, None)


def _norm(src, dst, curv, dinv):
    kern = pl.kernel(
        _norm_body,
        out_type=[
            jax.ShapeDtypeStruct((NCHUNK, C), jnp.float32),
            jax.ShapeDtypeStruct((NCHUNK, C), jnp.float32),
        ],
        mesh=plsc.VectorSubcoreMesh(core_axis_name="c", subcore_axis_name="s",
                                    num_cores=NC),
        scratch_types=[
            pltpu.VMEM_SHARED((NP,), jnp.float32),
            pltpu.VMEM((C,), jnp.int32),
            pltpu.VMEM((C,), jnp.int32),
            pltpu.VMEM((C,), jnp.float32),
            pltpu.VMEM((C,), jnp.float32),
            pltpu.VMEM((C,), jnp.float32),
            pltpu.VMEM((C,), jnp.float32),
            pltpu.VMEM((C,), jnp.float32),
        ],
    )
    return kern(src, dst, curv, dinv)


def _agg_body(tbl_hbm, src_hbm, dst_hbm, norm_hbm, ghet_hbm, zrows_hbm,
              g_out, d_out,
              gacc, dacc,
              srcv, dstv, gsrc, gdst, normv, ghetv,
              xs, xd, gbuf, dbuf, sem, sem2):
    cidx = lax.axis_index("c")
    sidx = lax.axis_index("s")
    base = sidx * STRIPE

    pltpu.sync_copy(zrows_hbm, gacc.at[pl.ds(base, STRIPE)])
    pltpu.sync_copy(zrows_hbm, dacc.at[pl.ds(base, STRIPE)])
    plsc.subcore_barrier()

    n_a = jnp.where(sidx < NCHUNK % NS, NCHUNK // NS + 1, NCHUNK // NS)

    def body(i, _):
        k = sidx + i * NS
        pltpu.sync_copy(src_hbm.at[k], srcv)
        pltpu.sync_copy(dst_hbm.at[k], dstv)
        pltpu.sync_copy(norm_hbm.at[k], normv)
        pltpu.sync_copy(ghet_hbm.at[k], ghetv)
        for v in range(RC):
            sl = pl.ds(v * L, L)
            gsrc[sl] = srcv[sl] * 2 + cidx
            gdst[sl] = dstv[sl] * 2 + cidx
        cp1 = pltpu.async_copy(tbl_hbm.at[gsrc], xs, sem)
        cp2 = pltpu.async_copy(tbl_hbm.at[gdst], xd, sem2)
        cp1.wait()
        cp2.wait()

        def ebody(v, _):
            nv = normv[pl.ds(v * L, L)]
            hv = ghetv[pl.ds(v * L, L)]
            for r in range(L):
                j = v * L + r
                nj = jnp.full((L,), nv[r], jnp.float32)
                hj = jnp.full((L,), hv[r], jnp.float32)
                for q in range(H // L):
                    sl = pl.ds(q * L, L)
                    a = xs[j, sl]
                    b = xd[j, sl]
                    gbuf[j, sl] = nj * a
                    dbuf[j, sl] = hj * jnp.abs(b - a)
            return _

        lax.fori_loop(0, RC, ebody, None)
        pltpu.sync_copy(gbuf, gacc.at[dstv], add=True)
        pltpu.sync_copy(dbuf, dacc.at[dstv], add=True)
        return _

    lax.fori_loop(0, n_a, body, None)
    plsc.subcore_barrier()

    pltpu.sync_copy(gacc.at[pl.ds(base, STRIPE)],
                    g_out.at[cidx, pl.ds(base, STRIPE)])
    pltpu.sync_copy(dacc.at[pl.ds(base, STRIPE)],
                    d_out.at[cidx, pl.ds(base, STRIPE)])


def _agg(tbl, src, dst, norm, ghet, zrows):
    kern = pl.kernel(
        _agg_body,
        out_type=[
            jax.ShapeDtypeStruct((NC, NP, H), jnp.float32),
            jax.ShapeDtypeStruct((NC, NP, H), jnp.float32),
        ],
        mesh=plsc.VectorSubcoreMesh(core_axis_name="c", subcore_axis_name="s",
                                    num_cores=NC),
        compiler_params=pltpu.CompilerParams(use_tc_tiling_on_sc=False),
        scratch_types=[
            pltpu.VMEM_SHARED((NP, H), jnp.float32),
            pltpu.VMEM_SHARED((NP, H), jnp.float32),
            pltpu.VMEM((C,), jnp.int32),
            pltpu.VMEM((C,), jnp.int32),
            pltpu.VMEM((C,), jnp.int32),
            pltpu.VMEM((C,), jnp.int32),
            pltpu.VMEM((C,), jnp.float32),
            pltpu.VMEM((C,), jnp.float32),
            pltpu.VMEM((C, H), jnp.float32),
            pltpu.VMEM((C, H), jnp.float32),
            pltpu.VMEM((C, H), jnp.float32),
            pltpu.VMEM((C, H), jnp.float32),
            pltpu.SemaphoreType.DMA,
            pltpu.SemaphoreType.DMA,
        ],
    )
    return kern(tbl, src, dst, norm, ghet, zrows)


def _dense(G, Dm, X, dinv_col, degh_col, Wg, bg, Wh, bh, relu):
    R = 2048

    def body(g0_ref, g1_ref, d0_ref, d1_ref, x_ref, di_ref, dh_ref,
             wg_ref, bg_ref, wh_ref, bh_ref, o_ref):
        g = jnp.concatenate([g0_ref[0], g1_ref[0]], axis=-1)
        d = jnp.concatenate([d0_ref[0], d1_ref[0]], axis=-1)
        di = di_ref[...]
        t = g + (di * di) * x_ref[...]
        acc = jnp.dot(t, wg_ref[...], preferred_element_type=jnp.float32)
        acc = acc + jnp.dot(d, wh_ref[...],
                            preferred_element_type=jnp.float32)
        acc = acc + bg_ref[...] + dh_ref[...] * bh_ref[...]
        if relu:
            acc = jnp.maximum(acc, 0.0)
        o_ref[...] = acc

    half = lambda c: pl.BlockSpec((1, R, H), lambda i: (c, i, 0))
    mat = lambda: pl.BlockSpec((R, D), lambda i: (i, 0))
    col = lambda: pl.BlockSpec((R, 1), lambda i: (i, 0))
    w128 = lambda: pl.BlockSpec((D, D), lambda i: (0, 0))
    b1 = lambda: pl.BlockSpec((1, D), lambda i: (0, 0))
    return pl.pallas_call(
        body,
        out_shape=jax.ShapeDtypeStruct((NP, D), jnp.float32),
        grid=(NP // R,),
        in_specs=[half(0), half(1), half(0), half(1), mat(), col(), col(),
                  w128(), b1(), w128(), b1()],
        out_specs=mat(),
    )(G, G, Dm, Dm, X, dinv_col, degh_col, Wg, bg.reshape(1, D),
      Wh, bh.reshape(1, D))


def kernel(x, edge_curvature, W_g1, b_g1, W_h1, b_h1, W_g2, b_g2, W_h2, b_h2,
           edge_index):
    src = edge_index[0].astype(jnp.int32).reshape(NCHUNK, C)
    dst = edge_index[1].astype(jnp.int32).reshape(NCHUNK, C)
    curv = edge_curvature.astype(jnp.float32).reshape(NCHUNK, C)
    zn = jnp.zeros((STRIPE,), jnp.float32)
    zrows = jnp.zeros((STRIPE, H), jnp.float32)
    x_p = jnp.pad(x, ((0, NP - N), (0, 0)))

    degg, degh = _deg(dst, curv, zn)
    dinv = _dinv(degg).reshape(NP)
    norm, ghet = _norm(src, dst, curv, dinv)
    dinv_col = dinv.reshape(NP, 1)
    degh_col = degh.reshape(NP, 1)

    g1, d1 = _agg(x_p.reshape(2 * NP, H), src, dst, norm, ghet, zrows)
    h = _dense(g1, d1, x_p, dinv_col, degh_col, W_g1, b_g1, W_h1, b_h1,
               relu=True)

    g2, d2 = _agg(h.reshape(2 * NP, H), src, dst, norm, ghet, zrows)
    out = _dense(g2, d2, h, dinv_col, degh_col, W_g2, b_g2, W_h2, b_h2,
                 relu=False)
    return out[:N]


# speedup vs baseline: 9.1463x; 1.5890x over previous
import jax
import jax.numpy as jnp
from jax import lax
from jax.experimental import pallas as pl
from jax.experimental.pallas import tpu as pltpu
from jax.experimental.pallas import tpu_sc as plsc

N = 10000
E = 320000
D = 128
H = 64
NP = 10240
NC = 2
NS = 16
NW = NC * NS
L = 16
C = 128
NCHUNK = E // C
RC = C // L
STRIPE = NP // NS
WSTRIPE = NP // NW


def _sigm(v):
    return 1.0 / (1.0 + jnp.exp(v * (-0.2)))


def _deg_body(dst_hbm, curv_hbm, zn_hbm,
              degg_hbm, degh_hbm,
              degg, degh,
              gbuf, hbuf, curvv, dstv, wbuf, wbuf2):
    cidx = lax.axis_index("c")
    sidx = lax.axis_index("s")
    w = sidx * NC + cidx
    base = sidx * STRIPE

    pltpu.sync_copy(zn_hbm, degg.at[pl.ds(base, STRIPE)])
    pltpu.sync_copy(zn_hbm, degh.at[pl.ds(base, STRIPE)])
    plsc.subcore_barrier()

    n_a = jnp.where(sidx < NCHUNK % NS, NCHUNK // NS + 1, NCHUNK // NS)

    def a_body(i, _):
        k = sidx + i * NS
        pltpu.sync_copy(dst_hbm.at[k], dstv)
        pltpu.sync_copy(curv_hbm.at[k], curvv)
        for v in range(RC):
            sl = pl.ds(v * L, L)
            g = _sigm(curvv[sl])
            gbuf[sl] = g
            hbuf[sl] = 1.0 - g
        pltpu.sync_copy(gbuf, degg.at[dstv], add=True)
        pltpu.sync_copy(hbuf, degh.at[dstv], add=True)
        return _

    lax.fori_loop(0, n_a, a_body, None)
    plsc.subcore_barrier()

    wbase = w * WSTRIPE
    pltpu.sync_copy(degg.at[pl.ds(wbase, WSTRIPE)], wbuf)
    pltpu.sync_copy(wbuf, degg_hbm.at[pl.ds(wbase, WSTRIPE)])
    pltpu.sync_copy(degh.at[pl.ds(wbase, WSTRIPE)], wbuf2)
    pltpu.sync_copy(wbuf2, degh_hbm.at[pl.ds(wbase, WSTRIPE)])


def _deg(dst, curv, zn):
    kern = pl.kernel(
        _deg_body,
        out_type=[
            jax.ShapeDtypeStruct((NP,), jnp.float32),
            jax.ShapeDtypeStruct((NP,), jnp.float32),
        ],
        mesh=plsc.VectorSubcoreMesh(core_axis_name="c", subcore_axis_name="s",
                                    num_cores=NC),
        scratch_types=[
            pltpu.VMEM_SHARED((NP,), jnp.float32),
            pltpu.VMEM_SHARED((NP,), jnp.float32),
            pltpu.VMEM((C,), jnp.float32),
            pltpu.VMEM((C,), jnp.float32),
            pltpu.VMEM((C,), jnp.float32),
            pltpu.VMEM((C,), jnp.int32),
            pltpu.VMEM((WSTRIPE,), jnp.float32),
            pltpu.VMEM((WSTRIPE,), jnp.float32),
        ],
    )
    return kern(dst, curv, zn)


def _dinv(degg):
    def body(d_ref, o_ref):
        o_ref[...] = lax.rsqrt(d_ref[...] + 1.0)

    return pl.pallas_call(
        body,
        out_shape=jax.ShapeDtypeStruct((NP // D, D), jnp.float32),
    )(degg.reshape(NP // D, D))


def _norm_body(src_hbm, dst_hbm, curv_hbm, dinv_hbm,
               norm_hbm, ghet_hbm,
               dinv_sp,
               srcv, dstv, curvv, normv, ghetv, dsb, ddb):
    cidx = lax.axis_index("c")
    sidx = lax.axis_index("s")
    w = sidx * NC + cidx
    base = sidx * STRIPE

    pltpu.sync_copy(dinv_hbm.at[pl.ds(base, STRIPE)],
                    dinv_sp.at[pl.ds(base, STRIPE)])
    plsc.subcore_barrier()
    n_c = jnp.where(w < NCHUNK % NW, NCHUNK // NW + 1, NCHUNK // NW)

    def c_body(i, _):
        k = w + i * NW
        pltpu.sync_copy(src_hbm.at[k], srcv)
        pltpu.sync_copy(dst_hbm.at[k], dstv)
        pltpu.sync_copy(curv_hbm.at[k], curvv)
        pltpu.sync_copy(dinv_sp.at[srcv], dsb)
        pltpu.sync_copy(dinv_sp.at[dstv], ddb)
        for v in range(RC):
            sl = pl.ds(v * L, L)
            g = _sigm(curvv[sl])
            normv[sl] = dsb[sl] * g * ddb[sl]
            ghetv[sl] = 1.0 - g
        pltpu.sync_copy(normv, norm_hbm.at[k])
        pltpu.sync_copy(ghetv, ghet_hbm.at[k])
        return _

    lax.fori_loop(0, n_c, c_body, None)


def _norm(src, dst, curv, dinv):
    kern = pl.kernel(
        _norm_body,
        out_type=[
            jax.ShapeDtypeStruct((NCHUNK, C), jnp.float32),
            jax.ShapeDtypeStruct((NCHUNK, C), jnp.float32),
        ],
        mesh=plsc.VectorSubcoreMesh(core_axis_name="c", subcore_axis_name="s",
                                    num_cores=NC),
        scratch_types=[
            pltpu.VMEM_SHARED((NP,), jnp.float32),
            pltpu.VMEM((C,), jnp.int32),
            pltpu.VMEM((C,), jnp.int32),
            pltpu.VMEM((C,), jnp.float32),
            pltpu.VMEM((C,), jnp.float32),
            pltpu.VMEM((C,), jnp.float32),
            pltpu.VMEM((C,), jnp.float32),
            pltpu.VMEM((C,), jnp.float32),
        ],
    )
    return kern(src, dst, curv, dinv)


def _agg_body(tbl_hbm, src_hbm, dst_hbm, norm_hbm, ghet_hbm, zrows_hbm,
              g_out, d_out,
              gacc, dacc,
              srcv0, dstv0, normv0, ghetv0, gsrc0, gdst0,
              srcv1, dstv1, normv1, ghetv1, gsrc1, gdst1,
              xs0, xd0, xs1, xd1,
              s0a, s0b, s1a, s1b):
    cidx = lax.axis_index("c")
    sidx = lax.axis_index("s")
    base = sidx * STRIPE

    pltpu.sync_copy(zrows_hbm, gacc.at[pl.ds(base, STRIPE)])
    pltpu.sync_copy(zrows_hbm, dacc.at[pl.ds(base, STRIPE)])
    plsc.subcore_barrier()

    n_a = jnp.where(sidx < NCHUNK % NS, NCHUNK // NS + 1, NCHUNK // NS)
    npair = (n_a + 1) // 2

    def load_idx(i, srcv, dstv, normv, ghetv, gsrc, gdst, xs, xd, sa, sb):
        k = jnp.where(i < n_a, sidx + i * NS, 0)
        pltpu.sync_copy(src_hbm.at[k], srcv)
        pltpu.sync_copy(dst_hbm.at[k], dstv)
        pltpu.sync_copy(norm_hbm.at[k], normv)
        pltpu.sync_copy(ghet_hbm.at[k], ghetv)
        for v in range(RC):
            sl = pl.ds(v * L, L)
            gsrc[sl] = srcv[sl] * 2 + cidx
            gdst[sl] = dstv[sl] * 2 + cidx
        pltpu.async_copy(tbl_hbm.at[gsrc], xs, sa)
        pltpu.async_copy(tbl_hbm.at[gdst], xd, sb)

    def compute_scatter(i, dstv, normv, ghetv, xs, xd):
        f = jnp.where(i < n_a, 1.0, 0.0).astype(jnp.float32)

        def ebody(v, _):
            nv = normv[pl.ds(v * L, L)] * f
            hv = ghetv[pl.ds(v * L, L)] * f
            for r in range(L):
                j = v * L + r
                nj = jnp.full((L,), nv[r], jnp.float32)
                hj = jnp.full((L,), hv[r], jnp.float32)
                for q in range(H // L):
                    sl = pl.ds(q * L, L)
                    a = xs[j, sl]
                    b = xd[j, sl]
                    xd[j, sl] = hj * jnp.abs(b - a)
                    xs[j, sl] = nj * a
            return _

        lax.fori_loop(0, RC, ebody, None)
        pltpu.sync_copy(xs, gacc.at[dstv], add=True)
        pltpu.sync_copy(xd, dacc.at[dstv], add=True)

    load_idx(jnp.int32(0), srcv0, dstv0, normv0, ghetv0, gsrc0, gdst0,
             xs0, xd0, s0a, s0b)

    def pair_body(j, _):
        load_idx(2 * j + 1, srcv1, dstv1, normv1, ghetv1, gsrc1, gdst1,
                 xs1, xd1, s1a, s1b)
        pltpu.make_async_copy(tbl_hbm.at[gsrc0], xs0, s0a).wait()
        pltpu.make_async_copy(tbl_hbm.at[gdst0], xd0, s0b).wait()
        compute_scatter(2 * j, dstv0, normv0, ghetv0, xs0, xd0)
        load_idx(2 * j + 2, srcv0, dstv0, normv0, ghetv0, gsrc0, gdst0,
                 xs0, xd0, s0a, s0b)
        pltpu.make_async_copy(tbl_hbm.at[gsrc1], xs1, s1a).wait()
        pltpu.make_async_copy(tbl_hbm.at[gdst1], xd1, s1b).wait()
        compute_scatter(2 * j + 1, dstv1, normv1, ghetv1, xs1, xd1)
        return _

    lax.fori_loop(0, npair, pair_body, None)
    pltpu.make_async_copy(tbl_hbm.at[gsrc0], xs0, s0a).wait()
    pltpu.make_async_copy(tbl_hbm.at[gdst0], xd0, s0b).wait()
    plsc.subcore_barrier()

    pltpu.sync_copy(gacc.at[pl.ds(base, STRIPE)],
                    g_out.at[cidx, pl.ds(base, STRIPE)])
    pltpu.sync_copy(dacc.at[pl.ds(base, STRIPE)],
                    d_out.at[cidx, pl.ds(base, STRIPE)])


def _agg(tbl, src, dst, norm, ghet, zrows):
    kern = pl.kernel(
        _agg_body,
        out_type=[
            jax.ShapeDtypeStruct((NC, NP, H), jnp.float32),
            jax.ShapeDtypeStruct((NC, NP, H), jnp.float32),
        ],
        mesh=plsc.VectorSubcoreMesh(core_axis_name="c", subcore_axis_name="s",
                                    num_cores=NC),
        compiler_params=pltpu.CompilerParams(use_tc_tiling_on_sc=False),
        scratch_types=[
            pltpu.VMEM_SHARED((NP, H), jnp.float32),
            pltpu.VMEM_SHARED((NP, H), jnp.float32),
            pltpu.VMEM((C,), jnp.int32),
            pltpu.VMEM((C,), jnp.int32),
            pltpu.VMEM((C,), jnp.float32),
            pltpu.VMEM((C,), jnp.float32),
            pltpu.VMEM((C,), jnp.int32),
            pltpu.VMEM((C,), jnp.int32),
            pltpu.VMEM((C,), jnp.int32),
            pltpu.VMEM((C,), jnp.int32),
            pltpu.VMEM((C,), jnp.float32),
            pltpu.VMEM((C,), jnp.float32),
            pltpu.VMEM((C,), jnp.int32),
            pltpu.VMEM((C,), jnp.int32),
            pltpu.VMEM((C, H), jnp.float32),
            pltpu.VMEM((C, H), jnp.float32),
            pltpu.VMEM((C, H), jnp.float32),
            pltpu.VMEM((C, H), jnp.float32),
            pltpu.SemaphoreType.DMA,
            pltpu.SemaphoreType.DMA,
            pltpu.SemaphoreType.DMA,
            pltpu.SemaphoreType.DMA,
        ],
    )
    return kern(tbl, src, dst, norm, ghet, zrows)


def _dense(G, Dm, X, dinv_col, degh_col, Wg, bg, Wh, bh, relu):
    R = 2048

    def body(g0_ref, g1_ref, d0_ref, d1_ref, x_ref, di_ref, dh_ref,
             wg_ref, bg_ref, wh_ref, bh_ref, o_ref):
        g = jnp.concatenate([g0_ref[0], g1_ref[0]], axis=-1)
        d = jnp.concatenate([d0_ref[0], d1_ref[0]], axis=-1)
        di = di_ref[...]
        t = g + (di * di) * x_ref[...]
        acc = jnp.dot(t, wg_ref[...], preferred_element_type=jnp.float32)
        acc = acc + jnp.dot(d, wh_ref[...],
                            preferred_element_type=jnp.float32)
        acc = acc + bg_ref[...] + dh_ref[...] * bh_ref[...]
        if relu:
            acc = jnp.maximum(acc, 0.0)
        o_ref[...] = acc

    half = lambda c: pl.BlockSpec((1, R, H), lambda i: (c, i, 0))
    mat = lambda: pl.BlockSpec((R, D), lambda i: (i, 0))
    col = lambda: pl.BlockSpec((R, 1), lambda i: (i, 0))
    w128 = lambda: pl.BlockSpec((D, D), lambda i: (0, 0))
    b1 = lambda: pl.BlockSpec((1, D), lambda i: (0, 0))
    return pl.pallas_call(
        body,
        out_shape=jax.ShapeDtypeStruct((NP, D), jnp.float32),
        grid=(NP // R,),
        in_specs=[half(0), half(1), half(0), half(1), mat(), col(), col(),
                  w128(), b1(), w128(), b1()],
        out_specs=mat(),
    )(G, G, Dm, Dm, X, dinv_col, degh_col, Wg, bg.reshape(1, D),
      Wh, bh.reshape(1, D))


def kernel(x, edge_curvature, W_g1, b_g1, W_h1, b_h1, W_g2, b_g2, W_h2, b_h2,
           edge_index):
    src = edge_index[0].astype(jnp.int32).reshape(NCHUNK, C)
    dst = edge_index[1].astype(jnp.int32).reshape(NCHUNK, C)
    curv = edge_curvature.astype(jnp.float32).reshape(NCHUNK, C)
    zn = jnp.zeros((STRIPE,), jnp.float32)
    zrows = jnp.zeros((STRIPE, H), jnp.float32)
    x_p = jnp.pad(x, ((0, NP - N), (0, 0)))

    degg, degh = _deg(dst, curv, zn)
    dinv = _dinv(degg).reshape(NP)
    norm, ghet = _norm(src, dst, curv, dinv)
    dinv_col = dinv.reshape(NP, 1)
    degh_col = degh.reshape(NP, 1)

    g1, d1 = _agg(x_p.reshape(2 * NP, H), src, dst, norm, ghet, zrows)
    h = _dense(g1, d1, x_p, dinv_col, degh_col, W_g1, b_g1, W_h1, b_h1,
               relu=True)

    g2, d2 = _agg(h.reshape(2 * NP, H), src, dst, norm, ghet, zrows)
    out = _dense(g2, d2, h, dinv_col, degh_col, W_g2, b_g2, W_h2, b_h2,
                 relu=False)
    return out[:N]


# speedup vs baseline: 10.9167x; 1.1936x over previous
import jax
import jax.numpy as jnp
from jax import lax
from jax.experimental import pallas as pl
from jax.experimental.pallas import tpu as pltpu
from jax.experimental.pallas import tpu_sc as plsc

N = 10000
E = 320000
D = 128
H = 64
NP = 10240
NC = 2
NS = 16
NW = NC * NS
L = 16
C = 128
NCHUNK = E // C
RC = C // L
STRIPE = NP // NS
WSTRIPE = NP // NW


def _sigm(v):
    return 1.0 / (1.0 + jnp.exp(v * (-0.2)))


def _deg_body(dst_hbm, curv_hbm, zn_hbm,
              degg_hbm, degh_hbm,
              degg, degh,
              dstv0, curvv0, dstv1, curvv1, gbuf, hbuf,
              sd0, sc0, sd1, sc1):
    cidx = lax.axis_index("c")
    sidx = lax.axis_index("s")
    w = sidx * NC + cidx
    base = sidx * STRIPE

    pltpu.sync_copy(zn_hbm, degg.at[pl.ds(base, STRIPE)])
    pltpu.sync_copy(zn_hbm, degh.at[pl.ds(base, STRIPE)])
    plsc.subcore_barrier()

    n_c = jnp.where(w < NCHUNK % NW, NCHUNK // NW + 1, NCHUNK // NW)
    npair = (n_c + 1) // 2

    def load(i, dstv, curvv, sd, sc):
        k = jnp.where(i < n_c, w + i * NW, 0)
        pltpu.async_copy(dst_hbm.at[k], dstv, sd)
        pltpu.async_copy(curv_hbm.at[k], curvv, sc)

    def compute(i, dstv, curvv):
        f = jnp.where(i < n_c, 1.0, 0.0).astype(jnp.float32)
        for v in range(RC):
            sl = pl.ds(v * L, L)
            g = _sigm(curvv[sl]) * f
            gbuf[sl] = g
            hbuf[sl] = f - g
        pltpu.sync_copy(gbuf, degg.at[dstv], add=True)
        pltpu.sync_copy(hbuf, degh.at[dstv], add=True)

    load(jnp.int32(0), dstv0, curvv0, sd0, sc0)

    def pair_body(j, _):
        load(2 * j + 1, dstv1, curvv1, sd1, sc1)
        pltpu.make_async_copy(dst_hbm.at[0], dstv0, sd0).wait()
        pltpu.make_async_copy(curv_hbm.at[0], curvv0, sc0).wait()
        compute(2 * j, dstv0, curvv0)
        load(2 * j + 2, dstv0, curvv0, sd0, sc0)
        pltpu.make_async_copy(dst_hbm.at[0], dstv1, sd1).wait()
        pltpu.make_async_copy(curv_hbm.at[0], curvv1, sc1).wait()
        compute(2 * j + 1, dstv1, curvv1)
        return _

    lax.fori_loop(0, npair, pair_body, None)
    pltpu.make_async_copy(dst_hbm.at[0], dstv0, sd0).wait()
    pltpu.make_async_copy(curv_hbm.at[0], curvv0, sc0).wait()
    plsc.subcore_barrier()

    pltpu.sync_copy(degg.at[pl.ds(base, STRIPE)],
                    degg_hbm.at[cidx, pl.ds(base, STRIPE)])
    pltpu.sync_copy(degh.at[pl.ds(base, STRIPE)],
                    degh_hbm.at[cidx, pl.ds(base, STRIPE)])


def _deg(dst, curv, zn):
    kern = pl.kernel(
        _deg_body,
        out_type=[
            jax.ShapeDtypeStruct((NC, NP), jnp.float32),
            jax.ShapeDtypeStruct((NC, NP), jnp.float32),
        ],
        mesh=plsc.VectorSubcoreMesh(core_axis_name="c", subcore_axis_name="s",
                                    num_cores=NC),
        scratch_types=[
            pltpu.VMEM_SHARED((NP,), jnp.float32),
            pltpu.VMEM_SHARED((NP,), jnp.float32),
            pltpu.VMEM((C,), jnp.int32),
            pltpu.VMEM((C,), jnp.float32),
            pltpu.VMEM((C,), jnp.int32),
            pltpu.VMEM((C,), jnp.float32),
            pltpu.VMEM((C,), jnp.float32),
            pltpu.VMEM((C,), jnp.float32),
            pltpu.SemaphoreType.DMA,
            pltpu.SemaphoreType.DMA,
            pltpu.SemaphoreType.DMA,
            pltpu.SemaphoreType.DMA,
        ],
    )
    return kern(dst, curv, zn)


def _dinv(degg, degh):
    def body(g_ref, h_ref, o_ref, ho_ref):
        g = g_ref[0] + g_ref[1]
        o_ref[...] = lax.rsqrt(g + 1.0)
        ho_ref[...] = h_ref[0] + h_ref[1]

    return pl.pallas_call(
        body,
        out_shape=[
            jax.ShapeDtypeStruct((NP // D, D), jnp.float32),
            jax.ShapeDtypeStruct((NP // D, D), jnp.float32),
        ],
    )(degg.reshape(NC, NP // D, D), degh.reshape(NC, NP // D, D))


def _norm_body(src_hbm, dst_hbm, curv_hbm, dinv_hbm,
               norm_hbm, ghet_hbm,
               dinv_sp,
               srcv0, dstv0, curvv0, srcv1, dstv1, curvv1,
               normv, ghetv, dsb, ddb,
               ss0, sd0, sc0, ss1, sd1, sc1):
    cidx = lax.axis_index("c")
    sidx = lax.axis_index("s")
    w = sidx * NC + cidx
    base = sidx * STRIPE

    pltpu.sync_copy(dinv_hbm.at[pl.ds(base, STRIPE)],
                    dinv_sp.at[pl.ds(base, STRIPE)])
    plsc.subcore_barrier()
    n_c = jnp.where(w < NCHUNK % NW, NCHUNK // NW + 1, NCHUNK // NW)
    npair = (n_c + 1) // 2

    def load(i, srcv, dstv, curvv, ss, sd, sc):
        k = jnp.where(i < n_c, w + i * NW, 0)
        pltpu.async_copy(src_hbm.at[k], srcv, ss)
        pltpu.async_copy(dst_hbm.at[k], dstv, sd)
        pltpu.async_copy(curv_hbm.at[k], curvv, sc)

    def compute(i, srcv, dstv, curvv):
        k = jnp.where(i < n_c, w + i * NW, 0)
        pltpu.sync_copy(dinv_sp.at[srcv], dsb)
        pltpu.sync_copy(dinv_sp.at[dstv], ddb)
        for v in range(RC):
            sl = pl.ds(v * L, L)
            g = _sigm(curvv[sl])
            normv[sl] = dsb[sl] * g * ddb[sl]
            ghetv[sl] = 1.0 - g
        pltpu.sync_copy(normv, norm_hbm.at[k])
        pltpu.sync_copy(ghetv, ghet_hbm.at[k])

    load(jnp.int32(0), srcv0, dstv0, curvv0, ss0, sd0, sc0)

    def pair_body(j, _):
        load(2 * j + 1, srcv1, dstv1, curvv1, ss1, sd1, sc1)
        pltpu.make_async_copy(src_hbm.at[0], srcv0, ss0).wait()
        pltpu.make_async_copy(dst_hbm.at[0], dstv0, sd0).wait()
        pltpu.make_async_copy(curv_hbm.at[0], curvv0, sc0).wait()
        compute(2 * j, srcv0, dstv0, curvv0)
        load(2 * j + 2, srcv0, dstv0, curvv0, ss0, sd0, sc0)
        pltpu.make_async_copy(src_hbm.at[0], srcv1, ss1).wait()
        pltpu.make_async_copy(dst_hbm.at[0], dstv1, sd1).wait()
        pltpu.make_async_copy(curv_hbm.at[0], curvv1, sc1).wait()
        compute(2 * j + 1, srcv1, dstv1, curvv1)
        return _

    lax.fori_loop(0, npair, pair_body, None)
    pltpu.make_async_copy(src_hbm.at[0], srcv0, ss0).wait()
    pltpu.make_async_copy(dst_hbm.at[0], dstv0, sd0).wait()
    pltpu.make_async_copy(curv_hbm.at[0], curvv0, sc0).wait()


def _norm(src, dst, curv, dinv):
    kern = pl.kernel(
        _norm_body,
        out_type=[
            jax.ShapeDtypeStruct((NCHUNK, C), jnp.float32),
            jax.ShapeDtypeStruct((NCHUNK, C), jnp.float32),
        ],
        mesh=plsc.VectorSubcoreMesh(core_axis_name="c", subcore_axis_name="s",
                                    num_cores=NC),
        scratch_types=[
            pltpu.VMEM_SHARED((NP,), jnp.float32),
            pltpu.VMEM((C,), jnp.int32),
            pltpu.VMEM((C,), jnp.int32),
            pltpu.VMEM((C,), jnp.float32),
            pltpu.VMEM((C,), jnp.int32),
            pltpu.VMEM((C,), jnp.int32),
            pltpu.VMEM((C,), jnp.float32),
            pltpu.VMEM((C,), jnp.float32),
            pltpu.VMEM((C,), jnp.float32),
            pltpu.VMEM((C,), jnp.float32),
            pltpu.VMEM((C,), jnp.float32),
            pltpu.SemaphoreType.DMA,
            pltpu.SemaphoreType.DMA,
            pltpu.SemaphoreType.DMA,
            pltpu.SemaphoreType.DMA,
            pltpu.SemaphoreType.DMA,
            pltpu.SemaphoreType.DMA,
        ],
    )
    return kern(src, dst, curv, dinv)


def _agg_body(tbl_hbm, src_hbm, dst_hbm, norm_hbm, ghet_hbm, zrows_hbm,
              g_out, d_out,
              gacc, dacc,
              srcv0, dstv0, normv0, ghetv0, gsrc0, gdst0,
              srcv1, dstv1, normv1, ghetv1, gsrc1, gdst1,
              xs0, xd0, xs1, xd1,
              s0a, s0b, s1a, s1b):
    cidx = lax.axis_index("c")
    sidx = lax.axis_index("s")
    base = sidx * STRIPE

    pltpu.sync_copy(zrows_hbm, gacc.at[pl.ds(base, STRIPE)])
    pltpu.sync_copy(zrows_hbm, dacc.at[pl.ds(base, STRIPE)])
    plsc.subcore_barrier()

    n_a = jnp.where(sidx < NCHUNK % NS, NCHUNK // NS + 1, NCHUNK // NS)
    npair = (n_a + 1) // 2

    def load_idx(i, srcv, dstv, normv, ghetv, gsrc, gdst, xs, xd, sa, sb):
        k = jnp.where(i < n_a, sidx + i * NS, 0)
        pltpu.sync_copy(src_hbm.at[k], srcv)
        pltpu.sync_copy(dst_hbm.at[k], dstv)
        pltpu.sync_copy(norm_hbm.at[k], normv)
        pltpu.sync_copy(ghet_hbm.at[k], ghetv)
        for v in range(RC):
            sl = pl.ds(v * L, L)
            gsrc[sl] = srcv[sl] * 2 + cidx
            gdst[sl] = dstv[sl] * 2 + cidx
        pltpu.async_copy(tbl_hbm.at[gsrc], xs, sa)
        pltpu.async_copy(tbl_hbm.at[gdst], xd, sb)

    def compute_scatter(i, dstv, normv, ghetv, xs, xd):
        f = jnp.where(i < n_a, 1.0, 0.0).astype(jnp.float32)

        def ebody(v, _):
            nv = normv[pl.ds(v * L, L)] * f
            hv = ghetv[pl.ds(v * L, L)] * f
            for r in range(L):
                j = v * L + r
                nj = jnp.full((L,), nv[r], jnp.float32)
                hj = jnp.full((L,), hv[r], jnp.float32)
                for q in range(H // L):
                    sl = pl.ds(q * L, L)
                    a = xs[j, sl]
                    b = xd[j, sl]
                    xd[j, sl] = hj * jnp.abs(b - a)
                    xs[j, sl] = nj * a
            return _

        lax.fori_loop(0, RC, ebody, None)
        pltpu.sync_copy(xs, gacc.at[dstv], add=True)
        pltpu.sync_copy(xd, dacc.at[dstv], add=True)

    load_idx(jnp.int32(0), srcv0, dstv0, normv0, ghetv0, gsrc0, gdst0,
             xs0, xd0, s0a, s0b)

    def pair_body(j, _):
        load_idx(2 * j + 1, srcv1, dstv1, normv1, ghetv1, gsrc1, gdst1,
                 xs1, xd1, s1a, s1b)
        pltpu.make_async_copy(tbl_hbm.at[gsrc0], xs0, s0a).wait()
        pltpu.make_async_copy(tbl_hbm.at[gdst0], xd0, s0b).wait()
        compute_scatter(2 * j, dstv0, normv0, ghetv0, xs0, xd0)
        load_idx(2 * j + 2, srcv0, dstv0, normv0, ghetv0, gsrc0, gdst0,
                 xs0, xd0, s0a, s0b)
        pltpu.make_async_copy(tbl_hbm.at[gsrc1], xs1, s1a).wait()
        pltpu.make_async_copy(tbl_hbm.at[gdst1], xd1, s1b).wait()
        compute_scatter(2 * j + 1, dstv1, normv1, ghetv1, xs1, xd1)
        return _

    lax.fori_loop(0, npair, pair_body, None)
    pltpu.make_async_copy(tbl_hbm.at[gsrc0], xs0, s0a).wait()
    pltpu.make_async_copy(tbl_hbm.at[gdst0], xd0, s0b).wait()
    plsc.subcore_barrier()

    pltpu.sync_copy(gacc.at[pl.ds(base, STRIPE)],
                    g_out.at[cidx, pl.ds(base, STRIPE)])
    pltpu.sync_copy(dacc.at[pl.ds(base, STRIPE)],
                    d_out.at[cidx, pl.ds(base, STRIPE)])


def _agg(tbl, src, dst, norm, ghet, zrows):
    kern = pl.kernel(
        _agg_body,
        out_type=[
            jax.ShapeDtypeStruct((NC, NP, H), jnp.float32),
            jax.ShapeDtypeStruct((NC, NP, H), jnp.float32),
        ],
        mesh=plsc.VectorSubcoreMesh(core_axis_name="c", subcore_axis_name="s",
                                    num_cores=NC),
        compiler_params=pltpu.CompilerParams(use_tc_tiling_on_sc=False),
        scratch_types=[
            pltpu.VMEM_SHARED((NP, H), jnp.float32),
            pltpu.VMEM_SHARED((NP, H), jnp.float32),
            pltpu.VMEM((C,), jnp.int32),
            pltpu.VMEM((C,), jnp.int32),
            pltpu.VMEM((C,), jnp.float32),
            pltpu.VMEM((C,), jnp.float32),
            pltpu.VMEM((C,), jnp.int32),
            pltpu.VMEM((C,), jnp.int32),
            pltpu.VMEM((C,), jnp.int32),
            pltpu.VMEM((C,), jnp.int32),
            pltpu.VMEM((C,), jnp.float32),
            pltpu.VMEM((C,), jnp.float32),
            pltpu.VMEM((C,), jnp.int32),
            pltpu.VMEM((C,), jnp.int32),
            pltpu.VMEM((C, H), jnp.float32),
            pltpu.VMEM((C, H), jnp.float32),
            pltpu.VMEM((C, H), jnp.float32),
            pltpu.VMEM((C, H), jnp.float32),
            pltpu.SemaphoreType.DMA,
            pltpu.SemaphoreType.DMA,
            pltpu.SemaphoreType.DMA,
            pltpu.SemaphoreType.DMA,
        ],
    )
    return kern(tbl, src, dst, norm, ghet, zrows)


def _dense(G, Dm, X, dinv_col, degh_col, Wg, bg, Wh, bh, relu):
    R = 2048

    def body(g0_ref, g1_ref, d0_ref, d1_ref, x_ref, di_ref, dh_ref,
             wg_ref, bg_ref, wh_ref, bh_ref, o_ref):
        g = jnp.concatenate([g0_ref[0], g1_ref[0]], axis=-1)
        d = jnp.concatenate([d0_ref[0], d1_ref[0]], axis=-1)
        di = di_ref[...]
        t = g + (di * di) * x_ref[...]
        acc = jnp.dot(t, wg_ref[...], preferred_element_type=jnp.float32)
        acc = acc + jnp.dot(d, wh_ref[...],
                            preferred_element_type=jnp.float32)
        acc = acc + bg_ref[...] + dh_ref[...] * bh_ref[...]
        if relu:
            acc = jnp.maximum(acc, 0.0)
        o_ref[...] = acc

    half = lambda c: pl.BlockSpec((1, R, H), lambda i: (c, i, 0))
    mat = lambda: pl.BlockSpec((R, D), lambda i: (i, 0))
    col = lambda: pl.BlockSpec((R, 1), lambda i: (i, 0))
    w128 = lambda: pl.BlockSpec((D, D), lambda i: (0, 0))
    b1 = lambda: pl.BlockSpec((1, D), lambda i: (0, 0))
    return pl.pallas_call(
        body,
        out_shape=jax.ShapeDtypeStruct((NP, D), jnp.float32),
        grid=(NP // R,),
        in_specs=[half(0), half(1), half(0), half(1), mat(), col(), col(),
                  w128(), b1(), w128(), b1()],
        out_specs=mat(),
    )(G, G, Dm, Dm, X, dinv_col, degh_col, Wg, bg.reshape(1, D),
      Wh, bh.reshape(1, D))


def kernel(x, edge_curvature, W_g1, b_g1, W_h1, b_h1, W_g2, b_g2, W_h2, b_h2,
           edge_index):
    src = edge_index[0].astype(jnp.int32).reshape(NCHUNK, C)
    dst = edge_index[1].astype(jnp.int32).reshape(NCHUNK, C)
    curv = edge_curvature.astype(jnp.float32).reshape(NCHUNK, C)
    zn = jnp.zeros((STRIPE,), jnp.float32)
    zrows = jnp.zeros((STRIPE, H), jnp.float32)
    x_p = jnp.pad(x, ((0, NP - N), (0, 0)))

    deggp, deghp = _deg(dst, curv, zn)
    dinv2d, degh2d = _dinv(deggp, deghp)
    dinv = dinv2d.reshape(NP)
    norm, ghet = _norm(src, dst, curv, dinv)
    dinv_col = dinv.reshape(NP, 1)
    degh_col = degh2d.reshape(NP, 1)

    g1, d1 = _agg(x_p.reshape(2 * NP, H), src, dst, norm, ghet, zrows)
    h = _dense(g1, d1, x_p, dinv_col, degh_col, W_g1, b_g1, W_h1, b_h1,
               relu=True)

    g2, d2 = _agg(h.reshape(2 * NP, H), src, dst, norm, ghet, zrows)
    out = _dense(g2, d2, h, dinv_col, degh_col, W_g2, b_g2, W_h2, b_h2,
                 relu=False)
    return out[:N]


# speedup vs baseline: 11.5398x; 1.0571x over previous
import jax
import jax.numpy as jnp
from jax import lax
from jax.experimental import pallas as pl
from jax.experimental.pallas import tpu as pltpu
from jax.experimental.pallas import tpu_sc as plsc

N = 10000
E = 320000
D = 128
H = 64
NP = 10240
NC = 2
NS = 16
NW = NC * NS
L = 16
C = 128
NCHUNK = E // C
RC = C // L
STRIPE = NP // NS
WSTRIPE = NP // NW


def _sigm(v):
    return 1.0 / (1.0 + jnp.exp(v * (-0.2)))


def _deg_body(dst_hbm, curv_hbm, zn_hbm,
              degg_hbm, degh_hbm,
              degg, degh,
              dstv0, curvv0, dstv1, curvv1, gbuf, hbuf,
              sd0, sc0, sd1, sc1):
    cidx = lax.axis_index("c")
    sidx = lax.axis_index("s")
    w = sidx * NC + cidx
    base = sidx * STRIPE

    pltpu.sync_copy(zn_hbm, degg.at[pl.ds(base, STRIPE)])
    pltpu.sync_copy(zn_hbm, degh.at[pl.ds(base, STRIPE)])
    plsc.subcore_barrier()

    n_c = jnp.where(w < NCHUNK % NW, NCHUNK // NW + 1, NCHUNK // NW)
    npair = (n_c + 1) // 2

    def load(i, dstv, curvv, sd, sc):
        k = jnp.where(i < n_c, w + i * NW, 0)
        pltpu.async_copy(dst_hbm.at[k], dstv, sd)
        pltpu.async_copy(curv_hbm.at[k], curvv, sc)

    def compute(i, dstv, curvv):
        f = jnp.where(i < n_c, 1.0, 0.0).astype(jnp.float32)
        for v in range(RC):
            sl = pl.ds(v * L, L)
            g = _sigm(curvv[sl]) * f
            gbuf[sl] = g
            hbuf[sl] = f - g
        pltpu.sync_copy(gbuf, degg.at[dstv], add=True)
        pltpu.sync_copy(hbuf, degh.at[dstv], add=True)

    load(jnp.int32(0), dstv0, curvv0, sd0, sc0)

    def pair_body(j, _):
        load(2 * j + 1, dstv1, curvv1, sd1, sc1)
        pltpu.make_async_copy(dst_hbm.at[0], dstv0, sd0).wait()
        pltpu.make_async_copy(curv_hbm.at[0], curvv0, sc0).wait()
        compute(2 * j, dstv0, curvv0)
        load(2 * j + 2, dstv0, curvv0, sd0, sc0)
        pltpu.make_async_copy(dst_hbm.at[0], dstv1, sd1).wait()
        pltpu.make_async_copy(curv_hbm.at[0], curvv1, sc1).wait()
        compute(2 * j + 1, dstv1, curvv1)
        return _

    lax.fori_loop(0, npair, pair_body, None)
    pltpu.make_async_copy(dst_hbm.at[0], dstv0, sd0).wait()
    pltpu.make_async_copy(curv_hbm.at[0], curvv0, sc0).wait()
    plsc.subcore_barrier()

    pltpu.sync_copy(degg.at[pl.ds(base, STRIPE)],
                    degg_hbm.at[cidx, pl.ds(base, STRIPE)])
    pltpu.sync_copy(degh.at[pl.ds(base, STRIPE)],
                    degh_hbm.at[cidx, pl.ds(base, STRIPE)])


def _deg(dst, curv, zn):
    kern = pl.kernel(
        _deg_body,
        out_type=[
            jax.ShapeDtypeStruct((NC, NP), jnp.float32),
            jax.ShapeDtypeStruct((NC, NP), jnp.float32),
        ],
        mesh=plsc.VectorSubcoreMesh(core_axis_name="c", subcore_axis_name="s",
                                    num_cores=NC),
        scratch_types=[
            pltpu.VMEM_SHARED((NP,), jnp.float32),
            pltpu.VMEM_SHARED((NP,), jnp.float32),
            pltpu.VMEM((C,), jnp.int32),
            pltpu.VMEM((C,), jnp.float32),
            pltpu.VMEM((C,), jnp.int32),
            pltpu.VMEM((C,), jnp.float32),
            pltpu.VMEM((C,), jnp.float32),
            pltpu.VMEM((C,), jnp.float32),
            pltpu.SemaphoreType.DMA,
            pltpu.SemaphoreType.DMA,
            pltpu.SemaphoreType.DMA,
            pltpu.SemaphoreType.DMA,
        ],
    )
    return kern(dst, curv, zn)


def _dinv(degg, degh):
    def body(g_ref, h_ref, o_ref, ho_ref):
        g = g_ref[0] + g_ref[1]
        o_ref[...] = lax.rsqrt(g + 1.0)
        ho_ref[...] = h_ref[0] + h_ref[1]

    return pl.pallas_call(
        body,
        out_shape=[
            jax.ShapeDtypeStruct((NP // D, D), jnp.float32),
            jax.ShapeDtypeStruct((NP // D, D), jnp.float32),
        ],
    )(degg.reshape(NC, NP // D, D), degh.reshape(NC, NP // D, D))


def _norm_body(src_hbm, dst_hbm, curv_hbm, dinv_hbm,
               norm_hbm, ghet_hbm,
               dinv_sp,
               srcv0, dstv0, curvv0, srcv1, dstv1, curvv1,
               normv, ghetv, dsb, ddb,
               ss0, sd0, sc0, ss1, sd1, sc1):
    cidx = lax.axis_index("c")
    sidx = lax.axis_index("s")
    w = sidx * NC + cidx
    base = sidx * STRIPE

    pltpu.sync_copy(dinv_hbm.at[pl.ds(base, STRIPE)],
                    dinv_sp.at[pl.ds(base, STRIPE)])
    plsc.subcore_barrier()
    n_c = jnp.where(w < NCHUNK % NW, NCHUNK // NW + 1, NCHUNK // NW)
    npair = (n_c + 1) // 2

    def load(i, srcv, dstv, curvv, ss, sd, sc):
        k = jnp.where(i < n_c, w + i * NW, 0)
        pltpu.async_copy(src_hbm.at[k], srcv, ss)
        pltpu.async_copy(dst_hbm.at[k], dstv, sd)
        pltpu.async_copy(curv_hbm.at[k], curvv, sc)

    def compute(i, srcv, dstv, curvv):
        k = jnp.where(i < n_c, w + i * NW, 0)
        pltpu.sync_copy(dinv_sp.at[srcv], dsb)
        pltpu.sync_copy(dinv_sp.at[dstv], ddb)
        for v in range(RC):
            sl = pl.ds(v * L, L)
            g = _sigm(curvv[sl])
            normv[sl] = dsb[sl] * g * ddb[sl]
            ghetv[sl] = 1.0 - g
        pltpu.sync_copy(normv, norm_hbm.at[k])
        pltpu.sync_copy(ghetv, ghet_hbm.at[k])

    load(jnp.int32(0), srcv0, dstv0, curvv0, ss0, sd0, sc0)

    def pair_body(j, _):
        load(2 * j + 1, srcv1, dstv1, curvv1, ss1, sd1, sc1)
        pltpu.make_async_copy(src_hbm.at[0], srcv0, ss0).wait()
        pltpu.make_async_copy(dst_hbm.at[0], dstv0, sd0).wait()
        pltpu.make_async_copy(curv_hbm.at[0], curvv0, sc0).wait()
        compute(2 * j, srcv0, dstv0, curvv0)
        load(2 * j + 2, srcv0, dstv0, curvv0, ss0, sd0, sc0)
        pltpu.make_async_copy(src_hbm.at[0], srcv1, ss1).wait()
        pltpu.make_async_copy(dst_hbm.at[0], dstv1, sd1).wait()
        pltpu.make_async_copy(curv_hbm.at[0], curvv1, sc1).wait()
        compute(2 * j + 1, srcv1, dstv1, curvv1)
        return _

    lax.fori_loop(0, npair, pair_body, None)
    pltpu.make_async_copy(src_hbm.at[0], srcv0, ss0).wait()
    pltpu.make_async_copy(dst_hbm.at[0], dstv0, sd0).wait()
    pltpu.make_async_copy(curv_hbm.at[0], curvv0, sc0).wait()


def _norm(src, dst, curv, dinv):
    kern = pl.kernel(
        _norm_body,
        out_type=[
            jax.ShapeDtypeStruct((NCHUNK, C), jnp.float32),
            jax.ShapeDtypeStruct((NCHUNK, C), jnp.float32),
        ],
        mesh=plsc.VectorSubcoreMesh(core_axis_name="c", subcore_axis_name="s",
                                    num_cores=NC),
        scratch_types=[
            pltpu.VMEM_SHARED((NP,), jnp.float32),
            pltpu.VMEM((C,), jnp.int32),
            pltpu.VMEM((C,), jnp.int32),
            pltpu.VMEM((C,), jnp.float32),
            pltpu.VMEM((C,), jnp.int32),
            pltpu.VMEM((C,), jnp.int32),
            pltpu.VMEM((C,), jnp.float32),
            pltpu.VMEM((C,), jnp.float32),
            pltpu.VMEM((C,), jnp.float32),
            pltpu.VMEM((C,), jnp.float32),
            pltpu.VMEM((C,), jnp.float32),
            pltpu.SemaphoreType.DMA,
            pltpu.SemaphoreType.DMA,
            pltpu.SemaphoreType.DMA,
            pltpu.SemaphoreType.DMA,
            pltpu.SemaphoreType.DMA,
            pltpu.SemaphoreType.DMA,
        ],
    )
    return kern(src, dst, curv, dinv)


def _agg_body(tbl_hbm, src_hbm, dst_hbm, curv_hbm, dinv_hbm, zrows_hbm,
              g_out, d_out,
              gacc, dacc, dinv_sp,
              srcv0, dstv0, curvv0, dsb0, ddb0, gsrc0, gdst0,
              srcv1, dstv1, curvv1, dsb1, ddb1, gsrc1, gdst1,
              xs0, xd0, xs1, xd1,
              s0a, s0b, s1a, s1b):
    cidx = lax.axis_index("c")
    sidx = lax.axis_index("s")
    base = sidx * STRIPE

    pltpu.sync_copy(zrows_hbm, gacc.at[pl.ds(base, STRIPE)])
    pltpu.sync_copy(zrows_hbm, dacc.at[pl.ds(base, STRIPE)])
    pltpu.sync_copy(dinv_hbm.at[pl.ds(base, STRIPE)],
                    dinv_sp.at[pl.ds(base, STRIPE)])
    plsc.subcore_barrier()

    n_a = jnp.where(sidx < NCHUNK % NS, NCHUNK // NS + 1, NCHUNK // NS)
    npair = (n_a + 1) // 2

    def load_idx(i, srcv, dstv, curvv, dsb, ddb, gsrc, gdst, xs, xd, sa, sb):
        k = jnp.where(i < n_a, sidx + i * NS, 0)
        pltpu.sync_copy(src_hbm.at[k], srcv)
        pltpu.sync_copy(dst_hbm.at[k], dstv)
        pltpu.sync_copy(curv_hbm.at[k], curvv)
        pltpu.sync_copy(dinv_sp.at[srcv], dsb)
        pltpu.sync_copy(dinv_sp.at[dstv], ddb)
        for v in range(RC):
            sl = pl.ds(v * L, L)
            gsrc[sl] = srcv[sl] * 2 + cidx
            gdst[sl] = dstv[sl] * 2 + cidx
        pltpu.async_copy(tbl_hbm.at[gsrc], xs, sa)
        pltpu.async_copy(tbl_hbm.at[gdst], xd, sb)

    def compute_scatter(i, dstv, curvv, dsb, ddb, xs, xd):
        f = jnp.where(i < n_a, 1.0, 0.0).astype(jnp.float32)

        def ebody(v, _):
            sl8 = pl.ds(v * L, L)
            g = _sigm(curvv[sl8])
            nv = dsb[sl8] * g * ddb[sl8] * f
            hv = (1.0 - g) * f
            for r in range(L):
                j = v * L + r
                nj = jnp.full((L,), nv[r], jnp.float32)
                hj = jnp.full((L,), hv[r], jnp.float32)
                for q in range(H // L):
                    sl = pl.ds(q * L, L)
                    a = xs[j, sl]
                    b = xd[j, sl]
                    xd[j, sl] = hj * jnp.abs(b - a)
                    xs[j, sl] = nj * a
            return _

        lax.fori_loop(0, RC, ebody, None)
        pltpu.sync_copy(xs, gacc.at[dstv], add=True)
        pltpu.sync_copy(xd, dacc.at[dstv], add=True)

    load_idx(jnp.int32(0), srcv0, dstv0, curvv0, dsb0, ddb0, gsrc0, gdst0,
             xs0, xd0, s0a, s0b)

    def pair_body(j, _):
        load_idx(2 * j + 1, srcv1, dstv1, curvv1, dsb1, ddb1, gsrc1, gdst1,
                 xs1, xd1, s1a, s1b)
        pltpu.make_async_copy(tbl_hbm.at[gsrc0], xs0, s0a).wait()
        pltpu.make_async_copy(tbl_hbm.at[gdst0], xd0, s0b).wait()
        compute_scatter(2 * j, dstv0, curvv0, dsb0, ddb0, xs0, xd0)
        load_idx(2 * j + 2, srcv0, dstv0, curvv0, dsb0, ddb0, gsrc0, gdst0,
                 xs0, xd0, s0a, s0b)
        pltpu.make_async_copy(tbl_hbm.at[gsrc1], xs1, s1a).wait()
        pltpu.make_async_copy(tbl_hbm.at[gdst1], xd1, s1b).wait()
        compute_scatter(2 * j + 1, dstv1, curvv1, dsb1, ddb1, xs1, xd1)
        return _

    lax.fori_loop(0, npair, pair_body, None)
    pltpu.make_async_copy(tbl_hbm.at[gsrc0], xs0, s0a).wait()
    pltpu.make_async_copy(tbl_hbm.at[gdst0], xd0, s0b).wait()
    plsc.subcore_barrier()

    pltpu.sync_copy(gacc.at[pl.ds(base, STRIPE)],
                    g_out.at[cidx, pl.ds(base, STRIPE)])
    pltpu.sync_copy(dacc.at[pl.ds(base, STRIPE)],
                    d_out.at[cidx, pl.ds(base, STRIPE)])


def _agg(tbl, src, dst, curv, dinv, zrows):
    kern = pl.kernel(
        _agg_body,
        out_type=[
            jax.ShapeDtypeStruct((NC, NP, H), jnp.float32),
            jax.ShapeDtypeStruct((NC, NP, H), jnp.float32),
        ],
        mesh=plsc.VectorSubcoreMesh(core_axis_name="c", subcore_axis_name="s",
                                    num_cores=NC),
        compiler_params=pltpu.CompilerParams(use_tc_tiling_on_sc=False),
        scratch_types=[
            pltpu.VMEM_SHARED((NP, H), jnp.float32),
            pltpu.VMEM_SHARED((NP, H), jnp.float32),
            pltpu.VMEM_SHARED((NP,), jnp.float32),
            pltpu.VMEM((C,), jnp.int32),
            pltpu.VMEM((C,), jnp.int32),
            pltpu.VMEM((C,), jnp.float32),
            pltpu.VMEM((C,), jnp.float32),
            pltpu.VMEM((C,), jnp.float32),
            pltpu.VMEM((C,), jnp.int32),
            pltpu.VMEM((C,), jnp.int32),
            pltpu.VMEM((C,), jnp.int32),
            pltpu.VMEM((C,), jnp.int32),
            pltpu.VMEM((C,), jnp.float32),
            pltpu.VMEM((C,), jnp.float32),
            pltpu.VMEM((C,), jnp.float32),
            pltpu.VMEM((C,), jnp.int32),
            pltpu.VMEM((C,), jnp.int32),
            pltpu.VMEM((C, H), jnp.float32),
            pltpu.VMEM((C, H), jnp.float32),
            pltpu.VMEM((C, H), jnp.float32),
            pltpu.VMEM((C, H), jnp.float32),
            pltpu.SemaphoreType.DMA,
            pltpu.SemaphoreType.DMA,
            pltpu.SemaphoreType.DMA,
            pltpu.SemaphoreType.DMA,
        ],
    )
    return kern(tbl, src, dst, curv, dinv, zrows)


def _dense(G, Dm, X, dinv_col, degh_col, Wg, bg, Wh, bh, relu):
    R = 2048

    def body(g0_ref, g1_ref, d0_ref, d1_ref, x_ref, di_ref, dh_ref,
             wg_ref, bg_ref, wh_ref, bh_ref, o_ref):
        g = jnp.concatenate([g0_ref[0], g1_ref[0]], axis=-1)
        d = jnp.concatenate([d0_ref[0], d1_ref[0]], axis=-1)
        di = di_ref[...]
        t = g + (di * di) * x_ref[...]
        acc = jnp.dot(t, wg_ref[...], preferred_element_type=jnp.float32)
        acc = acc + jnp.dot(d, wh_ref[...],
                            preferred_element_type=jnp.float32)
        acc = acc + bg_ref[...] + dh_ref[...] * bh_ref[...]
        if relu:
            acc = jnp.maximum(acc, 0.0)
        o_ref[...] = acc

    half = lambda c: pl.BlockSpec((1, R, H), lambda i: (c, i, 0))
    mat = lambda: pl.BlockSpec((R, D), lambda i: (i, 0))
    col = lambda: pl.BlockSpec((R, 1), lambda i: (i, 0))
    w128 = lambda: pl.BlockSpec((D, D), lambda i: (0, 0))
    b1 = lambda: pl.BlockSpec((1, D), lambda i: (0, 0))
    return pl.pallas_call(
        body,
        out_shape=jax.ShapeDtypeStruct((NP, D), jnp.float32),
        grid=(NP // R,),
        in_specs=[half(0), half(1), half(0), half(1), mat(), col(), col(),
                  w128(), b1(), w128(), b1()],
        out_specs=mat(),
    )(G, G, Dm, Dm, X, dinv_col, degh_col, Wg, bg.reshape(1, D),
      Wh, bh.reshape(1, D))


def kernel(x, edge_curvature, W_g1, b_g1, W_h1, b_h1, W_g2, b_g2, W_h2, b_h2,
           edge_index):
    src = edge_index[0].astype(jnp.int32).reshape(NCHUNK, C)
    dst = edge_index[1].astype(jnp.int32).reshape(NCHUNK, C)
    curv = edge_curvature.astype(jnp.float32).reshape(NCHUNK, C)
    zn = jnp.zeros((STRIPE,), jnp.float32)
    zrows = jnp.zeros((STRIPE, H), jnp.float32)
    x_p = jnp.pad(x, ((0, NP - N), (0, 0)))

    deggp, deghp = _deg(dst, curv, zn)
    dinv2d, degh2d = _dinv(deggp, deghp)
    dinv = dinv2d.reshape(NP)
    dinv_col = dinv.reshape(NP, 1)
    degh_col = degh2d.reshape(NP, 1)

    g1, d1 = _agg(x_p.reshape(2 * NP, H), src, dst, curv, dinv, zrows)
    h = _dense(g1, d1, x_p, dinv_col, degh_col, W_g1, b_g1, W_h1, b_h1,
               relu=True)

    g2, d2 = _agg(h.reshape(2 * NP, H), src, dst, curv, dinv, zrows)
    out = _dense(g2, d2, h, dinv_col, degh_col, W_g2, b_g2, W_h2, b_h2,
                 relu=False)
    return out[:N]
